# baseline (speedup 1.0000x reference)
import functools

import jax
import jax.numpy as jnp
from jax import lax
from jax.experimental import pallas as pl
from jax.experimental.pallas import tpu as pltpu
from jax.experimental.pallas import tpu_sc as plsc

N = 10000
D = 128
NC = 2
NS = 16
NW = NC * NS
CHUNK = 112
CPT = 90
PAD_SPREAD = 240
NPAD = 10240
RPT = NPAD // NS
ZR = 80
MMR = 1000


def _sc_mesh():
    return plsc.VectorSubcoreMesh(core_axis_name="c", subcore_axis_name="s")


@functools.partial(
    pl.kernel,
    out_type=jax.ShapeDtypeStruct((NC * NPAD,), jnp.float32),
    mesh=_sc_mesh(),
    scratch_types=[
        pltpu.VMEM((CPT, CHUNK), jnp.int32),
        pltpu.VMEM((CHUNK,), jnp.float32),
        pltpu.VMEM((RPT,), jnp.float32),
        pltpu.VMEM_SHARED((NPAD,), jnp.float32),
        pltpu.SemaphoreType.DMA,
    ],
)
def _sc_degree(dst_hbm, zeros_hbm, deg_hbm, idx_v, ones_v, stg_v, acc_sh, sem):
    c = lax.axis_index("c")
    s = lax.axis_index("s")
    w = s * NC + c
    for k in range(CHUNK // 16):
        ones_v[pl.ds(16 * k, 16)] = jnp.full((16,), 1.0, dtype=jnp.float32)
    pltpu.sync_copy(zeros_hbm.at[pl.ds(0, RPT)], stg_v)
    pltpu.sync_copy(stg_v, acc_sh.at[pl.ds(s * RPT, RPT)])
    pltpu.sync_copy(dst_hbm.at[w], idx_v)
    plsc.subcore_barrier()

    def fire(j, carry):
        pltpu.async_copy(ones_v, acc_sh.at[idx_v.at[j]], sem, add=True)
        return carry

    lax.fori_loop(0, CPT, fire, 0)

    def drain(j, carry):
        pltpu.make_async_copy(ones_v, acc_sh.at[idx_v.at[0]], sem).wait()
        return carry

    lax.fori_loop(0, CPT, drain, 0)
    plsc.subcore_barrier()
    pltpu.sync_copy(acc_sh.at[pl.ds(s * RPT, RPT)], stg_v)
    pltpu.sync_copy(stg_v, deg_hbm.at[pl.ds(c * NPAD + s * RPT, RPT)])


@functools.partial(
    pl.kernel,
    out_type=jax.ShapeDtypeStruct((NC, NPAD, D), jnp.float32),
    mesh=_sc_mesh(),
    scratch_types=[
        pltpu.VMEM((3, CHUNK), jnp.int32),
        pltpu.VMEM((4, CHUNK), jnp.int32),
        pltpu.VMEM((3, CHUNK, D), jnp.float32),
        pltpu.VMEM_SHARED((NPAD, D), jnp.float32),
        pltpu.SemaphoreType.DMA((3,)),
        pltpu.SemaphoreType.DMA((3,)),
        pltpu.SemaphoreType.DMA((3,)),
    ],
)
def _sc_scatter(hp_hbm, srcf_hbm, dstf_hbm, zeros_hbm, parts_hbm,
                sbuf, dring, rows, acc_sh, sem_g, sem_i, sem_s):
    c = lax.axis_index("c")
    s = lax.axis_index("s")
    w = s * NC + c
    stg0 = rows.at[0, pl.ds(0, ZR)]
    pltpu.sync_copy(zeros_hbm, stg0)
    for j in range(RPT // ZR):
        pltpu.async_copy(stg0, acc_sh.at[pl.ds(s * RPT + j * ZR, ZR)],
                         sem_s.at[0])
    for j in range(RPT // ZR):
        pltpu.make_async_copy(stg0, acc_sh.at[pl.ds(s * RPT, ZR)],
                              sem_s.at[0]).wait()
    plsc.subcore_barrier()

    base = w * CPT * CHUNK

    def src_at(j):
        return srcf_hbm.at[pl.ds(base + j * CHUNK, CHUNK)]

    def dst_at(j):
        return dstf_hbm.at[pl.ds(base + j * CHUNK, CHUNK)]

    for t in range(3):
        pltpu.sync_copy(src_at(t), sbuf.at[t])
        pltpu.sync_copy(dst_at(t), dring.at[t])
    pltpu.async_copy(hp_hbm.at[sbuf.at[0]], rows.at[0], sem_g.at[0])
    pltpu.async_copy(hp_hbm.at[sbuf.at[1]], rows.at[1], sem_g.at[1])

    def body(j, carry):
        k = lax.rem(j, 3)
        kg = lax.rem(j + 2, 3)
        m = lax.rem(j, 4)
        mp = lax.rem(j + 3, 4)
        pltpu.make_async_copy(hp_hbm.at[sbuf.at[0]], rows.at[0],
                              sem_g.at[k]).wait()
        pltpu.async_copy(rows.at[k], acc_sh.at[dring.at[m]], sem_s.at[k],
                         add=True)

        @pl.when(jnp.logical_and(j >= 1, j + 2 < CPT))
        def _():
            pltpu.make_async_copy(src_at(0), sbuf.at[0], sem_i.at[kg]).wait()
            pltpu.make_async_copy(src_at(0), sbuf.at[0], sem_i.at[kg]).wait()
            pltpu.make_async_copy(rows.at[0], acc_sh.at[dring.at[0]],
                                  sem_s.at[kg]).wait()

        @pl.when(j + 2 < CPT)
        def _():
            pltpu.async_copy(hp_hbm.at[sbuf.at[kg]], rows.at[kg],
                             sem_g.at[kg])

        @pl.when(j + 3 < CPT)
        def _():
            pltpu.async_copy(src_at(j + 3), sbuf.at[k], sem_i.at[k])
            pltpu.async_copy(dst_at(j + 3), dring.at[mp], sem_i.at[k])

        return carry

    lax.fori_loop(0, CPT, body, 0)
    for t in range(3):
        pltpu.make_async_copy(rows.at[0], acc_sh.at[dring.at[0]],
                              sem_s.at[t]).wait()
    plsc.subcore_barrier()
    stgs = (rows.at[0, pl.ds(0, ZR)], rows.at[1, pl.ds(0, ZR)])
    for j in range(RPT // ZR):
        buf = stgs[j % 2]
        sem = sem_s.at[j % 2]
        if j >= 2:
            pltpu.make_async_copy(buf, parts_hbm.at[c, pl.ds(s * RPT, ZR)],
                                  sem).wait()
        pltpu.sync_copy(acc_sh.at[pl.ds(s * RPT + j * ZR, ZR)], buf)
        pltpu.async_copy(buf, parts_hbm.at[c, pl.ds(s * RPT + j * ZR, ZR)],
                         sem)
    for t in range(2):
        pltpu.make_async_copy(stgs[t], parts_hbm.at[c, pl.ds(s * RPT, ZR)],
                              sem_s.at[t]).wait()


def _mm_body(x_ref, w_ref, deg_ref, hp_ref):
    deg = jnp.sum(deg_ref[...], axis=1, keepdims=True) + 1.0
    dinv = lax.rsqrt(deg)
    h = jnp.dot(x_ref[...], w_ref[...], preferred_element_type=jnp.float32)
    hp_ref[...] = h * dinv


def _tc_matmul(x, W, deg2t):
    return pl.pallas_call(
        _mm_body,
        grid=(N // MMR,),
        in_specs=[
            pl.BlockSpec((MMR, D), lambda i: (i, 0)),
            pl.BlockSpec((D, D), lambda i: (0, 0)),
            pl.BlockSpec((MMR, NC), lambda i: (i, 0)),
        ],
        out_specs=pl.BlockSpec((MMR, D), lambda i: (i, 0)),
        out_shape=jax.ShapeDtypeStruct((N, D), jnp.float32),
    )(x, W, deg2t)


def _comb_body(parts_ref, hp_ref, deg_ref, b_ref, out_ref):
    deg = jnp.sum(deg_ref[...], axis=1, keepdims=True) + 1.0
    dinv = lax.rsqrt(deg)
    out_ref[...] = (parts_ref[0] + parts_ref[1] + hp_ref[...]) * dinv + b_ref[...]


def _tc_combine(parts, hp, deg2t, b2):
    return pl.pallas_call(
        _comb_body,
        grid=(N // MMR,),
        in_specs=[
            pl.BlockSpec((NC, MMR, D), lambda i: (0, i, 0)),
            pl.BlockSpec((MMR, D), lambda i: (i, 0)),
            pl.BlockSpec((MMR, NC), lambda i: (i, 0)),
            pl.BlockSpec((1, D), lambda i: (0, 0)),
        ],
        out_specs=pl.BlockSpec((MMR, D), lambda i: (i, 0)),
        out_shape=jax.ShapeDtypeStruct((N, D), jnp.float32),
    )(parts, hp, deg2t, b2)


def kernel(x, edge_index, W, b):
    src = edge_index[0].astype(jnp.int32)
    dst = edge_index[1].astype(jnp.int32)
    e = src.shape[0]
    epad = NW * CPT * CHUNK
    npe = epad - e
    pad_ids = jnp.arange(npe, dtype=jnp.int32) % PAD_SPREAD
    srcp = jnp.concatenate([src, pad_ids])
    dstf = jnp.concatenate([dst, N + pad_ids])
    dst3 = dstf.reshape(NW, CPT, CHUNK)
    zeros1 = jnp.zeros((RPT,), jnp.float32)
    zeros2 = jnp.zeros((ZR, D), jnp.float32)

    deg2 = _sc_degree(dst3, zeros1).reshape(NC, NPAD)
    deg2t = deg2[:, :N].T
    hp = _tc_matmul(x, W, deg2t)
    parts = _sc_scatter(hp, srcp, dstf, zeros2)
    out = _tc_combine(parts, hp, deg2t, b.reshape(1, D))
    return out.reshape(1, N, D)


# speedup vs baseline: 1.0095x; 1.0095x over previous
import functools

import jax
import jax.numpy as jnp
from jax import lax
from jax.experimental import pallas as pl
from jax.experimental.pallas import tpu as pltpu
from jax.experimental.pallas import tpu_sc as plsc

N = 10000
D = 128
NC = 2
NS = 16
NW = NC * NS
CHUNK = 80
CPT = 125
NBUF = 4
DRING = NBUF + 1
PAD_SPREAD = 240
NPAD = 10240
RPT = NPAD // NS
ZR = 80
MMR = 1000


def _sc_mesh():
    return plsc.VectorSubcoreMesh(core_axis_name="c", subcore_axis_name="s")


@functools.partial(
    pl.kernel,
    out_type=jax.ShapeDtypeStruct((NC * NPAD,), jnp.float32),
    mesh=_sc_mesh(),
    scratch_types=[
        pltpu.VMEM((CPT, CHUNK), jnp.int32),
        pltpu.VMEM((CHUNK,), jnp.float32),
        pltpu.VMEM((RPT,), jnp.float32),
        pltpu.VMEM_SHARED((NPAD,), jnp.float32),
        pltpu.SemaphoreType.DMA,
    ],
)
def _sc_degree(dst_hbm, zeros_hbm, deg_hbm, idx_v, ones_v, stg_v, acc_sh, sem):
    c = lax.axis_index("c")
    s = lax.axis_index("s")
    w = s * NC + c
    for k in range(CHUNK // 16):
        ones_v[pl.ds(16 * k, 16)] = jnp.full((16,), 1.0, dtype=jnp.float32)
    pltpu.sync_copy(zeros_hbm.at[pl.ds(0, RPT)], stg_v)
    pltpu.sync_copy(stg_v, acc_sh.at[pl.ds(s * RPT, RPT)])
    pltpu.sync_copy(dst_hbm.at[w], idx_v)
    plsc.subcore_barrier()

    def fire(j, carry):
        pltpu.async_copy(ones_v, acc_sh.at[idx_v.at[j]], sem, add=True)
        return carry

    lax.fori_loop(0, CPT, fire, 0)

    def drain(j, carry):
        pltpu.make_async_copy(ones_v, acc_sh.at[idx_v.at[0]], sem).wait()
        return carry

    lax.fori_loop(0, CPT, drain, 0)
    plsc.subcore_barrier()
    pltpu.sync_copy(acc_sh.at[pl.ds(s * RPT, RPT)], stg_v)
    pltpu.sync_copy(stg_v, deg_hbm.at[pl.ds(c * NPAD + s * RPT, RPT)])


@functools.partial(
    pl.kernel,
    out_type=jax.ShapeDtypeStruct((NC, NPAD, D), jnp.float32),
    mesh=_sc_mesh(),
    scratch_types=[
        pltpu.VMEM((NBUF, CHUNK), jnp.int32),
        pltpu.VMEM((DRING, CHUNK), jnp.int32),
        pltpu.VMEM((NBUF, CHUNK, D), jnp.float32),
        pltpu.VMEM_SHARED((NPAD, D), jnp.float32),
        pltpu.SemaphoreType.DMA((NBUF,)),
        pltpu.SemaphoreType.DMA((NBUF,)),
        pltpu.SemaphoreType.DMA((NBUF,)),
    ],
)
def _sc_scatter(hp_hbm, srcf_hbm, dstf_hbm, zeros_hbm, parts_hbm,
                sbuf, dring, rows, acc_sh, sem_g, sem_i, sem_s):
    c = lax.axis_index("c")
    s = lax.axis_index("s")
    w = s * NC + c
    stg0 = rows.at[0, pl.ds(0, ZR)]
    pltpu.sync_copy(zeros_hbm, stg0)
    for j in range(RPT // ZR):
        pltpu.async_copy(stg0, acc_sh.at[pl.ds(s * RPT + j * ZR, ZR)],
                         sem_s.at[0])
    for j in range(RPT // ZR):
        pltpu.make_async_copy(stg0, acc_sh.at[pl.ds(s * RPT, ZR)],
                              sem_s.at[0]).wait()
    plsc.subcore_barrier()

    base = w * CPT * CHUNK

    def src_at(j):
        return srcf_hbm.at[pl.ds(base + j * CHUNK, CHUNK)]

    def dst_at(j):
        return dstf_hbm.at[pl.ds(base + j * CHUNK, CHUNK)]

    for t in range(NBUF):
        pltpu.sync_copy(src_at(t), sbuf.at[t])
        pltpu.sync_copy(dst_at(t), dring.at[t])
    for t in range(NBUF - 1):
        pltpu.async_copy(hp_hbm.at[sbuf.at[t]], rows.at[t], sem_g.at[t])

    G = NBUF - 1

    def body(j, carry):
        k = lax.rem(j, NBUF)
        kg = lax.rem(j + G, NBUF)
        m = lax.rem(j, DRING)
        mp = lax.rem(j + NBUF, DRING)
        pltpu.make_async_copy(hp_hbm.at[sbuf.at[0]], rows.at[0],
                              sem_g.at[k]).wait()
        pltpu.async_copy(rows.at[k], acc_sh.at[dring.at[m]], sem_s.at[k],
                         add=True)

        @pl.when(jnp.logical_and(j >= 1, j + G < CPT))
        def _():
            pltpu.make_async_copy(src_at(0), sbuf.at[0], sem_i.at[kg]).wait()
            pltpu.make_async_copy(src_at(0), sbuf.at[0], sem_i.at[kg]).wait()
            pltpu.make_async_copy(rows.at[0], acc_sh.at[dring.at[0]],
                                  sem_s.at[kg]).wait()

        @pl.when(j + G < CPT)
        def _():
            pltpu.async_copy(hp_hbm.at[sbuf.at[kg]], rows.at[kg],
                             sem_g.at[kg])

        @pl.when(j + NBUF < CPT)
        def _():
            pltpu.async_copy(src_at(j + NBUF), sbuf.at[k], sem_i.at[k])
            pltpu.async_copy(dst_at(j + NBUF), dring.at[mp], sem_i.at[k])

        return carry

    lax.fori_loop(0, CPT, body, 0)
    for t in range(NBUF):
        pltpu.make_async_copy(rows.at[0], acc_sh.at[dring.at[0]],
                              sem_s.at[t]).wait()
    plsc.subcore_barrier()
    stgs = (rows.at[0, pl.ds(0, ZR)], rows.at[1, pl.ds(0, ZR)])
    for j in range(RPT // ZR):
        buf = stgs[j % 2]
        sem = sem_s.at[j % 2]
        if j >= 2:
            pltpu.make_async_copy(buf, parts_hbm.at[c, pl.ds(s * RPT, ZR)],
                                  sem).wait()
        pltpu.sync_copy(acc_sh.at[pl.ds(s * RPT + j * ZR, ZR)], buf)
        pltpu.async_copy(buf, parts_hbm.at[c, pl.ds(s * RPT + j * ZR, ZR)],
                         sem)
    for t in range(2):
        pltpu.make_async_copy(stgs[t], parts_hbm.at[c, pl.ds(s * RPT, ZR)],
                              sem_s.at[t]).wait()


def _mm_body(x_ref, w_ref, deg_ref, hp_ref):
    deg = jnp.sum(deg_ref[...], axis=1, keepdims=True) + 1.0
    dinv = lax.rsqrt(deg)
    h = jnp.dot(x_ref[...], w_ref[...], preferred_element_type=jnp.float32)
    hp_ref[...] = h * dinv


def _tc_matmul(x, W, deg2t):
    return pl.pallas_call(
        _mm_body,
        grid=(N // MMR,),
        in_specs=[
            pl.BlockSpec((MMR, D), lambda i: (i, 0)),
            pl.BlockSpec((D, D), lambda i: (0, 0)),
            pl.BlockSpec((MMR, NC), lambda i: (i, 0)),
        ],
        out_specs=pl.BlockSpec((MMR, D), lambda i: (i, 0)),
        out_shape=jax.ShapeDtypeStruct((N, D), jnp.float32),
    )(x, W, deg2t)


def _comb_body(parts_ref, hp_ref, deg_ref, b_ref, out_ref):
    deg = jnp.sum(deg_ref[...], axis=1, keepdims=True) + 1.0
    dinv = lax.rsqrt(deg)
    out_ref[...] = (parts_ref[0] + parts_ref[1] + hp_ref[...]) * dinv + b_ref[...]


def _tc_combine(parts, hp, deg2t, b2):
    return pl.pallas_call(
        _comb_body,
        grid=(N // MMR,),
        in_specs=[
            pl.BlockSpec((NC, MMR, D), lambda i: (0, i, 0)),
            pl.BlockSpec((MMR, D), lambda i: (i, 0)),
            pl.BlockSpec((MMR, NC), lambda i: (i, 0)),
            pl.BlockSpec((1, D), lambda i: (0, 0)),
        ],
        out_specs=pl.BlockSpec((MMR, D), lambda i: (i, 0)),
        out_shape=jax.ShapeDtypeStruct((N, D), jnp.float32),
    )(parts, hp, deg2t, b2)


def kernel(x, edge_index, W, b):
    src = edge_index[0].astype(jnp.int32)
    dst = edge_index[1].astype(jnp.int32)
    e = src.shape[0]
    epad = NW * CPT * CHUNK
    npe = epad - e
    pad_ids = jnp.arange(npe, dtype=jnp.int32) % PAD_SPREAD
    srcp = jnp.concatenate([src, pad_ids])
    dstf = jnp.concatenate([dst, N + pad_ids])
    dst3 = dstf.reshape(NW, CPT, CHUNK)
    zeros1 = jnp.zeros((RPT,), jnp.float32)
    zeros2 = jnp.zeros((ZR, D), jnp.float32)

    deg2 = _sc_degree(dst3, zeros1).reshape(NC, NPAD)
    deg2t = deg2[:, :N].T
    hp = _tc_matmul(x, W, deg2t)
    parts = _sc_scatter(hp, srcp, dstf, zeros2)
    out = _tc_combine(parts, hp, deg2t, b.reshape(1, D))
    return out.reshape(1, N, D)
